# baseline (speedup 1.0000x reference)
import functools

import jax
import jax.numpy as jnp
from jax import lax
from jax.experimental import pallas as pl
from jax.experimental.pallas import tpu as pltpu
from jax.experimental.pallas import tpu_sc as plsc

N = 10000
E = 320000
D = 128
NLAYERS = 3
AF = 9
AV = 128
BCOMB = 512

NC, NS, LANES = 2, 16, 16
NW = NC * NS
DG = D // LANES

NPW = 320
N_PAD = NPW * NW
ACH = 32
EPW = E // NW
DEG_CH = 1000
ECH = 80
N_TBL = N_PAD
RPT = N_TBL // NS

f32 = jnp.float32
i32 = jnp.int32


def _dyn_gather(v, idx):
    dnums = lax.GatherDimensionNumbers(
        offset_dims=(), collapsed_slice_dims=(0,), start_index_map=(0,))
    return lax.gather(v, idx[:, None], dnums, slice_sizes=(1,),
                      mode=lax.GatherScatterMode.PROMISE_IN_BOUNDS)


def _bcast_lane(v, j):
    return _dyn_gather(v, jnp.full((LANES,), j, dtype=i32))


def _atom_deg_body(atab, xoff, rowe, h0p, degp,
                   xoi, abuf, hbuf, ones_b, rowi, zdeg, deg_s, sem):
    c = lax.axis_index("c")
    s = lax.axis_index("s")
    wid = s * NC + c
    tid = s

    nb0 = wid * NPW

    def atom_chunk(k, carry):
        base = nb0 + k * ACH
        for f in range(AF):
            pltpu.sync_copy(xoff.at[f, pl.ds(base, ACH)], xoi.at[f])
        cps = [pltpu.async_copy(atab.at[xoi.at[f]], abuf.at[f], sem)
               for f in range(AF)]
        for cp in cps:
            cp.wait()

        def rowloop(i, carry2):
            for d in range(DG):
                sl = pl.ds(d * LANES, LANES)
                acc = abuf[0, i, sl]
                for f in range(1, AF):
                    acc = acc + abuf[f, i, sl]
                hbuf[i, sl] = acc
            return carry2

        lax.fori_loop(0, ACH, rowloop, 0)
        pltpu.sync_copy(hbuf, h0p.at[pl.ds(base, ACH), :])
        return carry

    lax.fori_loop(0, NPW // ACH, atom_chunk, 0)

    def fill_ones(i, carry):
        ones_b[i, :] = jnp.full((LANES,), 1.0, dtype=f32)
        return carry

    lax.fori_loop(0, DEG_CH, fill_ones, 0)

    def fill_zero(i, carry):
        zdeg[i, :] = jnp.zeros((LANES,), dtype=f32)
        return carry

    lax.fori_loop(0, RPT, fill_zero, 0)
    pltpu.sync_copy(zdeg, deg_s.at[pl.ds(tid * RPT, RPT), :])
    plsc.subcore_barrier()

    eb0 = wid * EPW

    def deg_chunk(k, carry):
        pltpu.sync_copy(rowe.at[pl.ds(eb0 + k * DEG_CH, DEG_CH)], rowi)
        pltpu.sync_copy(ones_b, deg_s.at[rowi], add=True)
        return carry

    lax.fori_loop(0, EPW // DEG_CH, deg_chunk, 0)
    plsc.subcore_barrier()
    pltpu.sync_copy(deg_s.at[pl.ds(tid * RPT, RPT), :],
                    degp.at[c, pl.ds(tid * RPT, RPT), :])


_atom_deg_call = functools.partial(
    pl.kernel,
    out_type=(jax.ShapeDtypeStruct((N_PAD, D), f32),
              jax.ShapeDtypeStruct((NC, N_TBL, LANES), f32)),
    mesh=plsc.VectorSubcoreMesh(core_axis_name="c", subcore_axis_name="s"),
    compiler_params=pltpu.CompilerParams(use_tc_tiling_on_sc=False, needs_layout_passes=False),
    scratch_types=[
        pltpu.VMEM((AF, ACH), i32),
        pltpu.VMEM((AF, ACH, D), f32),
        pltpu.VMEM((ACH, D), f32),
        pltpu.VMEM((DEG_CH, LANES), f32),
        pltpu.VMEM((DEG_CH,), i32),
        pltpu.VMEM((RPT, LANES), f32),
        pltpu.VMEM_SHARED((N_TBL, LANES), f32),
        pltpu.SemaphoreType.DMA,
    ],
)(_atom_deg_body)


DH = D // NC
DGH = DH // LANES
EPS = E // NS
NCH = EPS // ECH

IROW, ICOL, IEIX = 0, 1, 2


def _edge_body(xlf, ipk, btabl2, dinv, aggp,
               i0, i1, ra0, ra1, xj0, xj1, ob0, ob1, btabv, dinvv, agg_s,
               isem0, isem1, g0, g1, setsem):
    c = lax.axis_index("c")
    s = lax.axis_index("s")
    tid = s
    cN = c * N

    cp_b = pltpu.async_copy(btabl2.at[c], btabv, setsem)
    cp_d = pltpu.async_copy(dinv, dinvv, setsem)

    pltpu.sync_copy(ipk.at[tid, 0], i0)
    pltpu.async_copy(ipk.at[tid, 1], i1, isem1)

    ZCH = 32

    def zrow(i, carry):
        for d in range(DGH):
            ob0[i, pl.ds(d * LANES, LANES)] = jnp.zeros((LANES,), dtype=f32)
        return carry

    lax.fori_loop(0, ZCH, zrow, 0)
    for k in range(RPT // ZCH):
        pltpu.sync_copy(ob0.at[pl.ds(0, ZCH), :],
                        agg_s.at[pl.ds(tid * RPT + k * ZCH, ZCH), :])
    plsc.subcore_barrier()
    cp_b.wait()
    cp_d.wait()

    def start_g(ib, ra, xjb, gsem):
        def adj(j, carry):
            sl = pl.ds(j * LANES, LANES)
            ra[sl] = ib[IROW, sl] + cN
            return carry

        lax.fori_loop(0, ECH // LANES, adj, 0)
        pltpu.async_copy(xlf.at[ra], xjb, gsem)

    iota16 = lax.iota(i32, LANES)

    def compute(ib, xjb, obb, gsem):
        pltpu.make_async_copy(xlf.at[pl.ds(0, ECH)], xjb, gsem).wait()

        def blk(j, carry):
            jb = j * LANES
            ev = ib[IEIX, pl.ds(jb, LANES)]
            rv = ib[IROW, pl.ds(jb, LANES)]
            dvv = plsc.load_gather(dinvv, [rv])
            evb = ev * DH
            for l in range(LANES):
                base = _bcast_lane(evb, l) + iota16
                dvb = _bcast_lane(dvv, l)
                i = jb + l
                for d in range(DGH):
                    sl = pl.ds(d * LANES, LANES)
                    bt = plsc.load_gather(btabv, [base + (d * LANES)])
                    obb[i, sl] = jnp.maximum(xjb[i, sl] + bt, 0.0) * dvb
            return carry

        lax.fori_loop(0, ECH // LANES, blk, 0)

    def scat(ib, obb):
        pltpu.sync_copy(obb, agg_s.at[ib.at[ICOL]], add=True)

    start_g(i0, ra0, xj0, g0)

    def pair(t, carry):
        k = 2 * t
        compute(i0, xj0, ob0, g0)
        pltpu.make_async_copy(ipk.at[0, 0], i1, isem1).wait()
        start_g(i1, ra1, xj1, g1)
        scat(i0, ob0)
        pltpu.async_copy(ipk.at[tid, k + 2], i0, isem0)
        compute(i1, xj1, ob1, g1)
        pltpu.make_async_copy(ipk.at[0, 0], i0, isem0).wait()
        start_g(i0, ra0, xj0, g0)
        scat(i1, ob1)

        @pl.when(k + 3 < NCH)
        def _():
            pltpu.async_copy(ipk.at[tid, k + 3], i1, isem1)

        return carry

    lax.fori_loop(0, (NCH - 2) // 2, pair, 0)
    compute(i0, xj0, ob0, g0)
    pltpu.make_async_copy(ipk.at[0, 0], i1, isem1).wait()
    start_g(i1, ra1, xj1, g1)
    scat(i0, ob0)
    compute(i1, xj1, ob1, g1)
    scat(i1, ob1)

    plsc.subcore_barrier()
    pltpu.sync_copy(agg_s.at[pl.ds(tid * RPT, RPT), :],
                    aggp.at[c, pl.ds(tid * RPT, RPT), :])


_edge_call = functools.partial(
    pl.kernel,
    out_type=jax.ShapeDtypeStruct((NC, N_TBL, DH), f32),
    mesh=plsc.VectorSubcoreMesh(core_axis_name="c", subcore_axis_name="s"),
    compiler_params=pltpu.CompilerParams(use_tc_tiling_on_sc=False, needs_layout_passes=False),
    scratch_types=[
        pltpu.VMEM((3, ECH), i32),
        pltpu.VMEM((3, ECH), i32),
        pltpu.VMEM((ECH,), i32),
        pltpu.VMEM((ECH,), i32),
        pltpu.VMEM((ECH, DH), f32),
        pltpu.VMEM((ECH, DH), f32),
        pltpu.VMEM((ECH, DH), f32),
        pltpu.VMEM((ECH, DH), f32),
        pltpu.VMEM((BCOMB * DH,), f32),
        pltpu.VMEM((N,), f32),
        pltpu.VMEM_SHARED((N_TBL, DH), f32),
        pltpu.SemaphoreType.DMA,
        pltpu.SemaphoreType.DMA,
        pltpu.SemaphoreType.DMA,
        pltpu.SemaphoreType.DMA,
        pltpu.SemaphoreType.DMA,
    ],
)(_edge_body)


def _btab_body(be_ref, out_ref):
    for l in range(NLAYERS):
        a = be_ref[l, 0]
        b_ = be_ref[l, 1]
        cc = be_ref[l, 2]
        u = (cc[:, None, :] + b_[None, :, :]).reshape(64, D)
        v = (u[:, None, :] + a[None, :, :]).reshape(BCOMB, D)
        out_ref[l] = v


_btab_call = pl.pallas_call(
    _btab_body,
    out_shape=jax.ShapeDtypeStruct((NLAYERS, BCOMB, D), f32),
)


def _prep_body(h0, w0, b0, degp, xl0, deg, dinv):
    d0 = degp[0, :, 0:1]
    d1 = degp[1, :, 0:1]
    degv = d0 + d1 + 1.0
    deg[...] = degv
    dinv[...] = lax.rsqrt(degv)
    xl0[...] = jnp.dot(h0[...], w0[...], preferred_element_type=f32) + b0[...]


_prep_call = pl.pallas_call(
    _prep_body,
    out_shape=(jax.ShapeDtypeStruct((N, D), f32),
               jax.ShapeDtypeStruct((N, 1), f32),
               jax.ShapeDtypeStruct((N, 1), f32)),
)


def _bn_core(aggp, xl, deg, dinv, root, gam, bet):
    agg = jnp.concatenate([aggp[0], aggp[1]], axis=-1)
    out = (agg * dinv[...]
           + jnp.maximum(xl[...] + root[...], 0.0) / deg[...])
    onesr = jnp.ones((1, N), dtype=f32)
    mean = jnp.dot(onesr, out, preferred_element_type=f32) / N
    sq = jnp.dot(onesr, out * out, preferred_element_type=f32) / N
    var = sq - mean * mean
    return gam[...] * (out - mean) * lax.rsqrt(var + 1e-5) + bet[...]


def _mid_layer_body(aggp, xl, deg, dinv, root, gam, bet, wn, bn, xln):
    hh = jnp.maximum(_bn_core(aggp, xl, deg, dinv, root, gam, bet), 0.0)
    xln[...] = jnp.dot(hh, wn[...], preferred_element_type=f32) + bn[...]


_mid_layer_call = pl.pallas_call(
    _mid_layer_body,
    out_shape=jax.ShapeDtypeStruct((N, D), f32),
)


def _last_layer_body(aggp, xl, deg, dinv, root, gam, bet, h_out):
    h_out[...] = _bn_core(aggp, xl, deg, dinv, root, gam, bet)


_last_layer_call = pl.pallas_call(
    _last_layer_body,
    out_shape=jax.ShapeDtypeStruct((N, D), f32),
)


def kernel(x, edge_index, edge_attr, atom_emb, bond_emb, W, b,
           root_emb, gamma, beta):
    x = x.astype(i32)
    ei = edge_index.astype(i32)
    ea = edge_attr.astype(i32)
    rowe = ei[0]
    cole = ei[1]
    xoff = x + (jnp.arange(AF, dtype=i32) * AV)[None, :]
    xoffT = jnp.pad(xoff.T, ((0, 0), (0, N_PAD - N)))
    eidx = ea[:, 0] + 8 * ea[:, 1] + 64 * ea[:, 2]
    atab = atom_emb.reshape(AF * AV, D)

    btab = _btab_call(bond_emb)
    h0p, degp = _atom_deg_call(atab, xoffT, rowe)
    h0 = h0p[:N]
    xl, deg, dinv2 = _prep_call(h0, W[0], b[0][None, :], degp[:, :N])
    dinv = dinv2.reshape(N)

    ipk = jnp.stack([rowe.reshape(NS, NCH, ECH),
                     cole.reshape(NS, NCH, ECH),
                     eidx.reshape(NS, NCH, ECH)], axis=2)

    btab2 = (btab.reshape(NLAYERS, BCOMB, NC, DH)
             .transpose(0, 2, 1, 3).reshape(NLAYERS, NC, BCOMB * DH))

    h = None
    for l in range(NLAYERS):
        xlf = jnp.concatenate([xl[:, :DH], xl[:, DH:]], axis=0)
        aggp = _edge_call(xlf, ipk, btab2[l], dinv)[:, :N]
        if l < NLAYERS - 1:
            xl = _mid_layer_call(aggp, xl, deg, dinv2, root_emb[l][None, :],
                                 gamma[l][None, :], beta[l][None, :],
                                 W[l + 1], b[l + 1][None, :])
        else:
            h = _last_layer_call(aggp, xl, deg, dinv2, root_emb[l][None, :],
                                 gamma[l][None, :], beta[l][None, :])
    return h


# speedup vs baseline: 1.0332x; 1.0332x over previous
import functools

import jax
import jax.numpy as jnp
from jax import lax
from jax.experimental import pallas as pl
from jax.experimental.pallas import tpu as pltpu
from jax.experimental.pallas import tpu_sc as plsc

N = 10000
E = 320000
D = 128
NLAYERS = 3
AF = 9
AV = 128
BCOMB = 512

NC, NS, LANES = 2, 16, 16
NW = NC * NS
DG = D // LANES

NPW = 320
N_PAD = NPW * NW
ACH = 32
EPW = E // NW
DEG_CH = 1000
ECH = 80
N_TBL = N_PAD
RPT = N_TBL // NS

f32 = jnp.float32
i32 = jnp.int32


def _dyn_gather(v, idx):
    dnums = lax.GatherDimensionNumbers(
        offset_dims=(), collapsed_slice_dims=(0,), start_index_map=(0,))
    return lax.gather(v, idx[:, None], dnums, slice_sizes=(1,),
                      mode=lax.GatherScatterMode.PROMISE_IN_BOUNDS)


def _bcast_lane(v, j):
    return _dyn_gather(v, jnp.full((LANES,), j, dtype=i32))


def _atom_deg_body(atab, xoff, rowe, h0p, degp,
                   xoi, abuf, hbuf, ones_b, rowi, zdeg, deg_s, sem):
    c = lax.axis_index("c")
    s = lax.axis_index("s")
    wid = s * NC + c
    tid = s

    nb0 = wid * NPW

    def atom_chunk(k, carry):
        base = nb0 + k * ACH
        for f in range(AF):
            pltpu.sync_copy(xoff.at[f, pl.ds(base, ACH)], xoi.at[f])
        cps = [pltpu.async_copy(atab.at[xoi.at[f]], abuf.at[f], sem)
               for f in range(AF)]
        for cp in cps:
            cp.wait()

        def rowloop(i, carry2):
            for d in range(DG):
                sl = pl.ds(d * LANES, LANES)
                acc = abuf[0, i, sl]
                for f in range(1, AF):
                    acc = acc + abuf[f, i, sl]
                hbuf[i, sl] = acc
            return carry2

        lax.fori_loop(0, ACH, rowloop, 0)
        pltpu.sync_copy(hbuf, h0p.at[pl.ds(base, ACH), :])
        return carry

    lax.fori_loop(0, NPW // ACH, atom_chunk, 0)

    def fill_ones(i, carry):
        ones_b[i, :] = jnp.full((LANES,), 1.0, dtype=f32)
        return carry

    lax.fori_loop(0, DEG_CH, fill_ones, 0)

    def fill_zero(i, carry):
        zdeg[i, :] = jnp.zeros((LANES,), dtype=f32)
        return carry

    lax.fori_loop(0, RPT, fill_zero, 0)
    pltpu.sync_copy(zdeg, deg_s.at[pl.ds(tid * RPT, RPT), :])
    plsc.subcore_barrier()

    eb0 = wid * EPW

    def deg_chunk(k, carry):
        pltpu.sync_copy(rowe.at[pl.ds(eb0 + k * DEG_CH, DEG_CH)], rowi)
        pltpu.sync_copy(ones_b, deg_s.at[rowi], add=True)
        return carry

    lax.fori_loop(0, EPW // DEG_CH, deg_chunk, 0)
    plsc.subcore_barrier()
    pltpu.sync_copy(deg_s.at[pl.ds(tid * RPT, RPT), :],
                    degp.at[c, pl.ds(tid * RPT, RPT), :])


_atom_deg_call = functools.partial(
    pl.kernel,
    out_type=(jax.ShapeDtypeStruct((N_PAD, D), f32),
              jax.ShapeDtypeStruct((NC, N_TBL, LANES), f32)),
    mesh=plsc.VectorSubcoreMesh(core_axis_name="c", subcore_axis_name="s"),
    compiler_params=pltpu.CompilerParams(use_tc_tiling_on_sc=False, needs_layout_passes=False),
    scratch_types=[
        pltpu.VMEM((AF, ACH), i32),
        pltpu.VMEM((AF, ACH, D), f32),
        pltpu.VMEM((ACH, D), f32),
        pltpu.VMEM((DEG_CH, LANES), f32),
        pltpu.VMEM((DEG_CH,), i32),
        pltpu.VMEM((RPT, LANES), f32),
        pltpu.VMEM_SHARED((N_TBL, LANES), f32),
        pltpu.SemaphoreType.DMA,
    ],
)(_atom_deg_body)


DE = D + LANES
NCH = EPW // ECH

IROW, ICOL, IEIX = 0, 1, 2


def _edge_body(xze, ipk, btabl, aggp,
               i0, i1, xj0, xj1, ob0, ob1, agg_s,
               isem0, isem1, g0, g1):
    c = lax.axis_index("c")
    s = lax.axis_index("s")
    wid = s * NC + c
    tid = s

    pltpu.sync_copy(ipk.at[wid, 0], i0)
    pltpu.async_copy(ipk.at[wid, 1], i1, isem1)

    ZCH = 32

    def zrow(i, carry):
        for d in range(DG):
            ob0[i, pl.ds(d * LANES, LANES)] = jnp.zeros((LANES,), dtype=f32)
        return carry

    lax.fori_loop(0, ZCH, zrow, 0)
    for k in range(RPT // ZCH):
        pltpu.sync_copy(ob0.at[pl.ds(0, ZCH), :],
                        agg_s.at[pl.ds(tid * RPT + k * ZCH, ZCH), :])
    plsc.subcore_barrier()

    def start_g(ib, xjb, obb, gsem):
        pltpu.async_copy(xze.at[ib.at[IROW]], xjb, gsem)
        pltpu.async_copy(btabl.at[ib.at[IEIX]], obb, gsem)

    def compute(xjb, obb, gsem):
        pltpu.make_async_copy(xze.at[pl.ds(0, ECH)], xjb, gsem).wait()
        pltpu.make_async_copy(btabl.at[pl.ds(0, ECH)], obb, gsem).wait()

        def edge(i, carry):
            nbv = xjb[i, pl.ds(D, LANES)]
            for d in range(DG):
                sl = pl.ds(d * LANES, LANES)
                obb[i, sl] = jnp.maximum(xjb[i, sl] + obb[i, sl], 0.0) * nbv
            return carry

        lax.fori_loop(0, ECH, edge, 0)

    def scat(ib, obb):
        pltpu.sync_copy(obb, agg_s.at[ib.at[ICOL]], add=True)

    start_g(i0, xj0, ob0, g0)

    def pair(t, carry):
        k = 2 * t
        compute(xj0, ob0, g0)
        pltpu.make_async_copy(ipk.at[0, 0], i1, isem1).wait()
        start_g(i1, xj1, ob1, g1)
        scat(i0, ob0)
        pltpu.async_copy(ipk.at[wid, k + 2], i0, isem0)
        compute(xj1, ob1, g1)
        pltpu.make_async_copy(ipk.at[0, 0], i0, isem0).wait()
        start_g(i0, xj0, ob0, g0)
        scat(i1, ob1)

        @pl.when(k + 3 < NCH)
        def _():
            pltpu.async_copy(ipk.at[wid, k + 3], i1, isem1)

        return carry

    lax.fori_loop(0, (NCH - 1) // 2, pair, 0)
    compute(xj0, ob0, g0)
    scat(i0, ob0)

    plsc.subcore_barrier()
    pltpu.sync_copy(agg_s.at[pl.ds(tid * RPT, RPT), :],
                    aggp.at[c, pl.ds(tid * RPT, RPT), :])


_edge_call = functools.partial(
    pl.kernel,
    out_type=jax.ShapeDtypeStruct((NC, N_TBL, D), f32),
    mesh=plsc.VectorSubcoreMesh(core_axis_name="c", subcore_axis_name="s"),
    compiler_params=pltpu.CompilerParams(use_tc_tiling_on_sc=False, needs_layout_passes=False),
    scratch_types=[
        pltpu.VMEM((3, ECH), i32),
        pltpu.VMEM((3, ECH), i32),
        pltpu.VMEM((ECH, DE), f32),
        pltpu.VMEM((ECH, DE), f32),
        pltpu.VMEM((ECH, D), f32),
        pltpu.VMEM((ECH, D), f32),
        pltpu.VMEM_SHARED((N_TBL, D), f32),
        pltpu.SemaphoreType.DMA,
        pltpu.SemaphoreType.DMA,
        pltpu.SemaphoreType.DMA,
        pltpu.SemaphoreType.DMA,
    ],
)(_edge_body)


def _btab_body(be_ref, out_ref):
    for l in range(NLAYERS):
        a = be_ref[l, 0]
        b_ = be_ref[l, 1]
        cc = be_ref[l, 2]
        u = (cc[:, None, :] + b_[None, :, :]).reshape(64, D)
        v = (u[:, None, :] + a[None, :, :]).reshape(BCOMB, D)
        out_ref[l] = v


_btab_call = pl.pallas_call(
    _btab_body,
    out_shape=jax.ShapeDtypeStruct((NLAYERS, BCOMB, D), f32),
)


def _prep_body(h0, w0, b0, degp, xl0, deg, dinv):
    d0 = degp[0, :, 0:1]
    d1 = degp[1, :, 0:1]
    degv = d0 + d1 + 1.0
    deg[...] = degv
    dinv[...] = lax.rsqrt(degv)
    xl0[...] = jnp.dot(h0[...], w0[...], preferred_element_type=f32) + b0[...]


_prep_call = pl.pallas_call(
    _prep_body,
    out_shape=(jax.ShapeDtypeStruct((N, D), f32),
               jax.ShapeDtypeStruct((N, 1), f32),
               jax.ShapeDtypeStruct((N, 1), f32)),
)


def _bn_core(aggp, xl, deg, dinv, root, gam, bet):
    out = ((aggp[0] + aggp[1]) * dinv[...]
           + jnp.maximum(xl[...] + root[...], 0.0) / deg[...])
    onesr = jnp.ones((1, N), dtype=f32)
    mean = jnp.dot(onesr, out, preferred_element_type=f32) / N
    sq = jnp.dot(onesr, out * out, preferred_element_type=f32) / N
    var = sq - mean * mean
    return gam[...] * (out - mean) * lax.rsqrt(var + 1e-5) + bet[...]


def _mid_layer_body(aggp, xl, deg, dinv, root, gam, bet, wn, bn, xln):
    hh = jnp.maximum(_bn_core(aggp, xl, deg, dinv, root, gam, bet), 0.0)
    xln[...] = jnp.dot(hh, wn[...], preferred_element_type=f32) + bn[...]


_mid_layer_call = pl.pallas_call(
    _mid_layer_body,
    out_shape=jax.ShapeDtypeStruct((N, D), f32),
)


def _last_layer_body(aggp, xl, deg, dinv, root, gam, bet, h_out):
    h_out[...] = _bn_core(aggp, xl, deg, dinv, root, gam, bet)


_last_layer_call = pl.pallas_call(
    _last_layer_body,
    out_shape=jax.ShapeDtypeStruct((N, D), f32),
)


def kernel(x, edge_index, edge_attr, atom_emb, bond_emb, W, b,
           root_emb, gamma, beta):
    x = x.astype(i32)
    ei = edge_index.astype(i32)
    ea = edge_attr.astype(i32)
    rowe = ei[0]
    cole = ei[1]
    xoff = x + (jnp.arange(AF, dtype=i32) * AV)[None, :]
    xoffT = jnp.pad(xoff.T, ((0, 0), (0, N_PAD - N)))
    eidx = ea[:, 0] + 8 * ea[:, 1] + 64 * ea[:, 2]
    atab = atom_emb.reshape(AF * AV, D)

    btab = _btab_call(bond_emb)
    h0p, degp = _atom_deg_call(atab, xoffT, rowe)
    h0 = h0p[:N]
    xl, deg, dinv2 = _prep_call(h0, W[0], b[0][None, :], degp[:, :N])

    ipk = jnp.stack([rowe.reshape(NW, NCH, ECH),
                     cole.reshape(NW, NCH, ECH),
                     eidx.reshape(NW, NCH, ECH)], axis=2)

    h = None
    for l in range(NLAYERS):
        xze = jnp.concatenate(
            [xl, jnp.broadcast_to(dinv2, (N, LANES))], axis=1)
        aggp = _edge_call(xze, ipk, btab[l])[:, :N]
        if l < NLAYERS - 1:
            xl = _mid_layer_call(aggp, xl, deg, dinv2, root_emb[l][None, :],
                                 gamma[l][None, :], beta[l][None, :],
                                 W[l + 1], b[l + 1][None, :])
        else:
            h = _last_layer_call(aggp, xl, deg, dinv2, root_emb[l][None, :],
                                 gamma[l][None, :], beta[l][None, :])
    return h


# speedup vs baseline: 1.9508x; 1.8881x over previous
import functools

import jax
import jax.numpy as jnp
from jax import lax
from jax.experimental import pallas as pl
from jax.experimental.pallas import tpu as pltpu
from jax.experimental.pallas import tpu_sc as plsc

N = 10000
E = 320000
D = 128
NLAYERS = 3
AF = 9
AV = 128
BCOMB = 512

NC, NS, LANES = 2, 16, 16
NW = NC * NS
DG = D // LANES

NPW = 320
N_PAD = NPW * NW
ACH = 32
EPW = E // NW
DEG_CH = 1000
ECH = 80
N_TBL = N_PAD
RPT = N_TBL // NS

f32 = jnp.float32
i32 = jnp.int32


def _dyn_gather(v, idx):
    dnums = lax.GatherDimensionNumbers(
        offset_dims=(), collapsed_slice_dims=(0,), start_index_map=(0,))
    return lax.gather(v, idx[:, None], dnums, slice_sizes=(1,),
                      mode=lax.GatherScatterMode.PROMISE_IN_BOUNDS)


def _bcast_lane(v, j):
    return _dyn_gather(v, jnp.full((LANES,), j, dtype=i32))


def _atom_deg_body(atab, xoff, rowe, h0p, degp,
                   xoi, abuf, hbuf, ones_b, rowi, zdeg, deg_s, sem):
    c = lax.axis_index("c")
    s = lax.axis_index("s")
    wid = s * NC + c
    tid = s

    nb0 = wid * NPW

    def atom_chunk(k, carry):
        base = nb0 + k * ACH
        for f in range(AF):
            pltpu.sync_copy(xoff.at[f, pl.ds(base, ACH)], xoi.at[f])
        cps = [pltpu.async_copy(atab.at[xoi.at[f]], abuf.at[f], sem)
               for f in range(AF)]
        for cp in cps:
            cp.wait()

        def rowloop(i, carry2):
            for d in range(DG):
                sl = pl.ds(d * LANES, LANES)
                acc = abuf[0, i, sl]
                for f in range(1, AF):
                    acc = acc + abuf[f, i, sl]
                hbuf[i, sl] = acc
            return carry2

        lax.fori_loop(0, ACH, rowloop, 0)
        pltpu.sync_copy(hbuf, h0p.at[pl.ds(base, ACH), :])
        return carry

    lax.fori_loop(0, NPW // ACH, atom_chunk, 0)

    def fill_ones(i, carry):
        ones_b[i, :] = jnp.full((LANES,), 1.0, dtype=f32)
        return carry

    lax.fori_loop(0, DEG_CH, fill_ones, 0)

    def fill_zero(i, carry):
        zdeg[i, :] = jnp.zeros((LANES,), dtype=f32)
        return carry

    lax.fori_loop(0, RPT, fill_zero, 0)
    pltpu.sync_copy(zdeg, deg_s.at[pl.ds(tid * RPT, RPT), :])
    plsc.subcore_barrier()

    eb0 = wid * EPW

    def deg_chunk(k, carry):
        pltpu.sync_copy(rowe.at[pl.ds(eb0 + k * DEG_CH, DEG_CH)], rowi)
        pltpu.sync_copy(ones_b, deg_s.at[rowi], add=True)
        return carry

    lax.fori_loop(0, EPW // DEG_CH, deg_chunk, 0)
    plsc.subcore_barrier()
    pltpu.sync_copy(deg_s.at[pl.ds(tid * RPT, RPT), :],
                    degp.at[c, pl.ds(tid * RPT, RPT), :])


_atom_deg_call = functools.partial(
    pl.kernel,
    out_type=(jax.ShapeDtypeStruct((N_PAD, D), f32),
              jax.ShapeDtypeStruct((NC, N_TBL, LANES), f32)),
    mesh=plsc.VectorSubcoreMesh(core_axis_name="c", subcore_axis_name="s"),
    compiler_params=pltpu.CompilerParams(use_tc_tiling_on_sc=False, needs_layout_passes=False),
    scratch_types=[
        pltpu.VMEM((AF, ACH), i32),
        pltpu.VMEM((AF, ACH, D), f32),
        pltpu.VMEM((ACH, D), f32),
        pltpu.VMEM((DEG_CH, LANES), f32),
        pltpu.VMEM((DEG_CH,), i32),
        pltpu.VMEM((RPT, LANES), f32),
        pltpu.VMEM_SHARED((N_TBL, LANES), f32),
        pltpu.SemaphoreType.DMA,
    ],
)(_atom_deg_body)


NCH = EPW // ECH

IROW, ICOL, IEIX = 0, 1, 2


def _edge_body(xl, ipk, btabl, dinvR, aggp,
               i0, i1, xj0, xj1, ob0, ob1, nb0, nb1, agg_s,
               isem0, isem1, g0, g1):
    c = lax.axis_index("c")
    s = lax.axis_index("s")
    wid = s * NC + c
    tid = s

    pltpu.sync_copy(ipk.at[wid, 0], i0)
    pltpu.async_copy(ipk.at[wid, 1], i1, isem1)

    ZCH = 32

    def zrow(i, carry):
        for d in range(DG):
            ob0[i, pl.ds(d * LANES, LANES)] = jnp.zeros((LANES,), dtype=f32)
        return carry

    lax.fori_loop(0, ZCH, zrow, 0)
    for k in range(RPT // ZCH):
        pltpu.sync_copy(ob0.at[pl.ds(0, ZCH), :],
                        agg_s.at[pl.ds(tid * RPT + k * ZCH, ZCH), :])
    plsc.subcore_barrier()

    def start_g(ib, xjb, obb, nbb, gsem):
        pltpu.async_copy(xl.at[ib.at[IROW]], xjb, gsem)
        pltpu.async_copy(btabl.at[ib.at[IEIX]], obb, gsem)
        pltpu.async_copy(dinvR.at[ib.at[IROW]], nbb, gsem)

    def compute(xjb, obb, nbb, gsem):
        pltpu.make_async_copy(xl.at[pl.ds(0, ECH)], xjb, gsem).wait()
        pltpu.make_async_copy(btabl.at[pl.ds(0, ECH)], obb, gsem).wait()
        pltpu.make_async_copy(dinvR.at[pl.ds(0, ECH)], nbb, gsem).wait()

        def edge(i, carry):
            nbv = nbb[i]
            for d in range(DG):
                sl = pl.ds(d * LANES, LANES)
                obb[i, sl] = jnp.maximum(xjb[i, sl] + obb[i, sl], 0.0) * nbv
            return carry

        lax.fori_loop(0, ECH, edge, 0)

    def scat(ib, obb):
        pltpu.sync_copy(obb, agg_s.at[ib.at[ICOL]], add=True)

    start_g(i0, xj0, ob0, nb0, g0)

    def pair(t, carry):
        k = 2 * t
        compute(xj0, ob0, nb0, g0)
        pltpu.make_async_copy(ipk.at[0, 0], i1, isem1).wait()
        start_g(i1, xj1, ob1, nb1, g1)
        scat(i0, ob0)
        pltpu.async_copy(ipk.at[wid, k + 2], i0, isem0)
        compute(xj1, ob1, nb1, g1)
        pltpu.make_async_copy(ipk.at[0, 0], i0, isem0).wait()
        start_g(i0, xj0, ob0, nb0, g0)
        scat(i1, ob1)

        @pl.when(k + 3 < NCH)
        def _():
            pltpu.async_copy(ipk.at[wid, k + 3], i1, isem1)

        return carry

    lax.fori_loop(0, (NCH - 1) // 2, pair, 0)
    compute(xj0, ob0, nb0, g0)
    scat(i0, ob0)

    plsc.subcore_barrier()
    pltpu.sync_copy(agg_s.at[pl.ds(tid * RPT, RPT), :],
                    aggp.at[c, pl.ds(tid * RPT, RPT), :])


_edge_call = functools.partial(
    pl.kernel,
    out_type=jax.ShapeDtypeStruct((NC, N_TBL, D), f32),
    mesh=plsc.VectorSubcoreMesh(core_axis_name="c", subcore_axis_name="s"),
    compiler_params=pltpu.CompilerParams(use_tc_tiling_on_sc=False, needs_layout_passes=False),
    scratch_types=[
        pltpu.VMEM((3, ECH), i32),
        pltpu.VMEM((3, ECH), i32),
        pltpu.VMEM((ECH, D), f32),
        pltpu.VMEM((ECH, D), f32),
        pltpu.VMEM((ECH, D), f32),
        pltpu.VMEM((ECH, D), f32),
        pltpu.VMEM((ECH, LANES), f32),
        pltpu.VMEM((ECH, LANES), f32),
        pltpu.VMEM_SHARED((N_TBL, D), f32),
        pltpu.SemaphoreType.DMA,
        pltpu.SemaphoreType.DMA,
        pltpu.SemaphoreType.DMA,
        pltpu.SemaphoreType.DMA,
    ],
)(_edge_body)


def _btab_body(be_ref, out_ref):
    for l in range(NLAYERS):
        a = be_ref[l, 0]
        b_ = be_ref[l, 1]
        cc = be_ref[l, 2]
        u = (cc[:, None, :] + b_[None, :, :]).reshape(64, D)
        v = (u[:, None, :] + a[None, :, :]).reshape(BCOMB, D)
        out_ref[l] = v


_btab_call = pl.pallas_call(
    _btab_body,
    out_shape=jax.ShapeDtypeStruct((NLAYERS, BCOMB, D), f32),
)


def _prep_body(h0, w0, b0, degp, xl0, deg, dinv):
    d0 = degp[0, :, 0:1]
    d1 = degp[1, :, 0:1]
    degv = d0 + d1 + 1.0
    deg[...] = degv
    dinv[...] = lax.rsqrt(degv)
    xl0[...] = jnp.dot(h0[...], w0[...], preferred_element_type=f32) + b0[...]


_prep_call = pl.pallas_call(
    _prep_body,
    out_shape=(jax.ShapeDtypeStruct((N, D), f32),
               jax.ShapeDtypeStruct((N, 1), f32),
               jax.ShapeDtypeStruct((N, 1), f32)),
)


def _bn_core(aggp, xl, deg, dinv, root, gam, bet):
    out = ((aggp[0] + aggp[1]) * dinv[...]
           + jnp.maximum(xl[...] + root[...], 0.0) / deg[...])
    onesr = jnp.ones((1, N), dtype=f32)
    mean = jnp.dot(onesr, out, preferred_element_type=f32) / N
    sq = jnp.dot(onesr, out * out, preferred_element_type=f32) / N
    var = sq - mean * mean
    return gam[...] * (out - mean) * lax.rsqrt(var + 1e-5) + bet[...]


def _mid_layer_body(aggp, xl, deg, dinv, root, gam, bet, wn, bn, xln):
    hh = jnp.maximum(_bn_core(aggp, xl, deg, dinv, root, gam, bet), 0.0)
    xln[...] = jnp.dot(hh, wn[...], preferred_element_type=f32) + bn[...]


_mid_layer_call = pl.pallas_call(
    _mid_layer_body,
    out_shape=jax.ShapeDtypeStruct((N, D), f32),
)


def _last_layer_body(aggp, xl, deg, dinv, root, gam, bet, h_out):
    h_out[...] = _bn_core(aggp, xl, deg, dinv, root, gam, bet)


_last_layer_call = pl.pallas_call(
    _last_layer_body,
    out_shape=jax.ShapeDtypeStruct((N, D), f32),
)


def kernel(x, edge_index, edge_attr, atom_emb, bond_emb, W, b,
           root_emb, gamma, beta):
    x = x.astype(i32)
    ei = edge_index.astype(i32)
    ea = edge_attr.astype(i32)
    rowe = ei[0]
    cole = ei[1]
    xoff = x + (jnp.arange(AF, dtype=i32) * AV)[None, :]
    xoffT = jnp.pad(xoff.T, ((0, 0), (0, N_PAD - N)))
    eidx = ea[:, 0] + 8 * ea[:, 1] + 64 * ea[:, 2]
    atab = atom_emb.reshape(AF * AV, D)

    btab = _btab_call(bond_emb)
    h0p, degp = _atom_deg_call(atab, xoffT, rowe)
    h0 = h0p[:N]
    xl, deg, dinv2 = _prep_call(h0, W[0], b[0][None, :], degp[:, :N])

    ipk = jnp.stack([rowe.reshape(NW, NCH, ECH),
                     cole.reshape(NW, NCH, ECH),
                     eidx.reshape(NW, NCH, ECH)], axis=2)

    dinvR = jnp.broadcast_to(dinv2, (N, LANES))

    h = None
    for l in range(NLAYERS):
        aggp = _edge_call(xl, ipk, btab[l], dinvR)[:, :N]
        if l < NLAYERS - 1:
            xl = _mid_layer_call(aggp, xl, deg, dinv2, root_emb[l][None, :],
                                 gamma[l][None, :], beta[l][None, :],
                                 W[l + 1], b[l + 1][None, :])
        else:
            h = _last_layer_call(aggp, xl, deg, dinv2, root_emb[l][None, :],
                                 gamma[l][None, :], beta[l][None, :])
    return h


# speedup vs baseline: 2.0056x; 1.0281x over previous
import functools

import jax
import jax.numpy as jnp
from jax import lax
from jax.experimental import pallas as pl
from jax.experimental.pallas import tpu as pltpu
from jax.experimental.pallas import tpu_sc as plsc

N = 10000
E = 320000
D = 128
NLAYERS = 3
AF = 9
AV = 128
BCOMB = 512

NC, NS, LANES = 2, 16, 16
NW = NC * NS
DG = D // LANES

NPW = 320
N_PAD = NPW * NW
ACH = 32
EPW = E // NW
DEG_CH = 1000
ECH = 80
N_TBL = N_PAD
RPT = N_TBL // NS

f32 = jnp.float32
i32 = jnp.int32


def _dyn_gather(v, idx):
    dnums = lax.GatherDimensionNumbers(
        offset_dims=(), collapsed_slice_dims=(0,), start_index_map=(0,))
    return lax.gather(v, idx[:, None], dnums, slice_sizes=(1,),
                      mode=lax.GatherScatterMode.PROMISE_IN_BOUNDS)


def _bcast_lane(v, j):
    return _dyn_gather(v, jnp.full((LANES,), j, dtype=i32))


def _atom_deg_body(atab, xoff, rowe, h0p, degp,
                   xoi, abufA, abufB, hbuf, ones_b, rowi, zdeg, deg_s,
                   semA, semB):
    c = lax.axis_index("c")
    s = lax.axis_index("s")
    wid = s * NC + c
    tid = s

    nb0 = wid * NPW

    pltpu.sync_copy(xoff.at[:, pl.ds(nb0, NPW)], xoi)

    def start_a(k, ab, asem):
        for f in range(AF):
            pltpu.async_copy(atab.at[xoi.at[f, pl.ds(k * ACH, ACH)]],
                             ab.at[f], asem)

    def wait_a(ab, asem):
        for f in range(AF):
            pltpu.make_async_copy(atab.at[pl.ds(0, ACH)], ab.at[f],
                                  asem).wait()

    def comp_a(k, ab):
        def rowloop(i, carry2):
            for d in range(DG):
                sl = pl.ds(d * LANES, LANES)
                acc = ab[0, i, sl]
                for f in range(1, AF):
                    acc = acc + ab[f, i, sl]
                hbuf[i, sl] = acc
            return carry2

        lax.fori_loop(0, ACH, rowloop, 0)
        pltpu.sync_copy(hbuf, h0p.at[pl.ds(nb0 + k * ACH, ACH), :])

    NA = NPW // ACH
    start_a(0, abufA, semA)

    def apair(t, carry):
        k = 2 * t
        start_a(k + 1, abufB, semB)
        wait_a(abufA, semA)
        comp_a(k, abufA)

        @pl.when(k + 2 < NA)
        def _():
            start_a(k + 2, abufA, semA)

        wait_a(abufB, semB)
        comp_a(k + 1, abufB)
        return carry

    lax.fori_loop(0, NA // 2, apair, 0)

    def fill_ones(i, carry):
        ones_b[i, :] = jnp.full((LANES,), 1.0, dtype=f32)
        return carry

    lax.fori_loop(0, DEG_CH, fill_ones, 0)

    def fill_zero(i, carry):
        zdeg[i, :] = jnp.zeros((LANES,), dtype=f32)
        return carry

    lax.fori_loop(0, RPT, fill_zero, 0)
    pltpu.sync_copy(zdeg, deg_s.at[pl.ds(tid * RPT, RPT), :])
    plsc.subcore_barrier()

    eb0 = wid * EPW

    def deg_chunk(k, carry):
        pltpu.sync_copy(rowe.at[pl.ds(eb0 + k * DEG_CH, DEG_CH)], rowi)
        pltpu.sync_copy(ones_b, deg_s.at[rowi], add=True)
        return carry

    lax.fori_loop(0, EPW // DEG_CH, deg_chunk, 0)
    plsc.subcore_barrier()
    pltpu.sync_copy(deg_s.at[pl.ds(tid * RPT, RPT), :],
                    degp.at[c, pl.ds(tid * RPT, RPT), :])


_atom_deg_call = functools.partial(
    pl.kernel,
    out_type=(jax.ShapeDtypeStruct((N_PAD, D), f32),
              jax.ShapeDtypeStruct((NC, N_TBL, LANES), f32)),
    mesh=plsc.VectorSubcoreMesh(core_axis_name="c", subcore_axis_name="s"),
    compiler_params=pltpu.CompilerParams(use_tc_tiling_on_sc=False, needs_layout_passes=False),
    scratch_types=[
        pltpu.VMEM((AF, NPW), i32),
        pltpu.VMEM((AF, ACH, D), f32),
        pltpu.VMEM((AF, ACH, D), f32),
        pltpu.VMEM((ACH, D), f32),
        pltpu.VMEM((DEG_CH, LANES), f32),
        pltpu.VMEM((DEG_CH,), i32),
        pltpu.VMEM((RPT, LANES), f32),
        pltpu.VMEM_SHARED((N_TBL, LANES), f32),
        pltpu.SemaphoreType.DMA,
        pltpu.SemaphoreType.DMA,
    ],
)(_atom_deg_body)


NCH = EPW // ECH

IROW, ICOL, IEIX = 0, 1, 2


def _edge_body(xl, ipk, btabl, dinvR, aggp,
               i0, i1, xj0, xj1, ob0, ob1, nb0, nb1, agg_s,
               isem0, isem1, g0, g1):
    c = lax.axis_index("c")
    s = lax.axis_index("s")
    wid = s * NC + c
    tid = s

    pltpu.sync_copy(ipk.at[wid, 0], i0)
    pltpu.async_copy(ipk.at[wid, 1], i1, isem1)

    ZCH = 32

    def zrow(i, carry):
        for d in range(DG):
            ob0[i, pl.ds(d * LANES, LANES)] = jnp.zeros((LANES,), dtype=f32)
        return carry

    lax.fori_loop(0, ZCH, zrow, 0)
    for k in range(RPT // ZCH):
        pltpu.sync_copy(ob0.at[pl.ds(0, ZCH), :],
                        agg_s.at[pl.ds(tid * RPT + k * ZCH, ZCH), :])
    plsc.subcore_barrier()

    def start_g(ib, xjb, obb, nbb, gsem):
        pltpu.async_copy(xl.at[ib.at[IROW]], xjb, gsem)
        pltpu.async_copy(btabl.at[ib.at[IEIX]], obb, gsem)
        pltpu.async_copy(dinvR.at[ib.at[IROW]], nbb, gsem)

    def compute(xjb, obb, nbb, gsem):
        pltpu.make_async_copy(xl.at[pl.ds(0, ECH)], xjb, gsem).wait()
        pltpu.make_async_copy(btabl.at[pl.ds(0, ECH)], obb, gsem).wait()
        pltpu.make_async_copy(dinvR.at[pl.ds(0, ECH)], nbb, gsem).wait()

        def edge(i, carry):
            nbv = nbb[i]
            for d in range(DG):
                sl = pl.ds(d * LANES, LANES)
                obb[i, sl] = jnp.maximum(xjb[i, sl] + obb[i, sl], 0.0) * nbv
            return carry

        lax.fori_loop(0, ECH, edge, 0)

    def scat(ib, obb):
        pltpu.sync_copy(obb, agg_s.at[ib.at[ICOL]], add=True)

    start_g(i0, xj0, ob0, nb0, g0)

    def pair(t, carry):
        k = 2 * t
        compute(xj0, ob0, nb0, g0)
        pltpu.make_async_copy(ipk.at[0, 0], i1, isem1).wait()
        start_g(i1, xj1, ob1, nb1, g1)
        scat(i0, ob0)
        pltpu.async_copy(ipk.at[wid, k + 2], i0, isem0)
        compute(xj1, ob1, nb1, g1)
        pltpu.make_async_copy(ipk.at[0, 0], i0, isem0).wait()
        start_g(i0, xj0, ob0, nb0, g0)
        scat(i1, ob1)

        @pl.when(k + 3 < NCH)
        def _():
            pltpu.async_copy(ipk.at[wid, k + 3], i1, isem1)

        return carry

    lax.fori_loop(0, (NCH - 1) // 2, pair, 0)
    compute(xj0, ob0, nb0, g0)
    scat(i0, ob0)

    plsc.subcore_barrier()
    pltpu.sync_copy(agg_s.at[pl.ds(tid * RPT, RPT), :],
                    aggp.at[c, pl.ds(tid * RPT, RPT), :])


_edge_call = functools.partial(
    pl.kernel,
    out_type=jax.ShapeDtypeStruct((NC, N_TBL, D), f32),
    mesh=plsc.VectorSubcoreMesh(core_axis_name="c", subcore_axis_name="s"),
    compiler_params=pltpu.CompilerParams(use_tc_tiling_on_sc=False, needs_layout_passes=False),
    scratch_types=[
        pltpu.VMEM((3, ECH), i32),
        pltpu.VMEM((3, ECH), i32),
        pltpu.VMEM((ECH, D), f32),
        pltpu.VMEM((ECH, D), f32),
        pltpu.VMEM((ECH, D), f32),
        pltpu.VMEM((ECH, D), f32),
        pltpu.VMEM((ECH, LANES), f32),
        pltpu.VMEM((ECH, LANES), f32),
        pltpu.VMEM_SHARED((N_TBL, D), f32),
        pltpu.SemaphoreType.DMA,
        pltpu.SemaphoreType.DMA,
        pltpu.SemaphoreType.DMA,
        pltpu.SemaphoreType.DMA,
    ],
)(_edge_body)


def _btab_body(be_ref, out_ref):
    for l in range(NLAYERS):
        a = be_ref[l, 0]
        b_ = be_ref[l, 1]
        cc = be_ref[l, 2]
        u = (cc[:, None, :] + b_[None, :, :]).reshape(64, D)
        v = (u[:, None, :] + a[None, :, :]).reshape(BCOMB, D)
        out_ref[l] = v


_btab_call = pl.pallas_call(
    _btab_body,
    out_shape=jax.ShapeDtypeStruct((NLAYERS, BCOMB, D), f32),
)


def _prep_body(h0, w0, b0, degp, xl0, deg, dinv):
    d0 = degp[0, :, 0:1]
    d1 = degp[1, :, 0:1]
    degv = d0 + d1 + 1.0
    deg[...] = degv
    dinv[...] = lax.rsqrt(degv)
    xl0[...] = jnp.dot(h0[...], w0[...], preferred_element_type=f32) + b0[...]


_prep_call = pl.pallas_call(
    _prep_body,
    out_shape=(jax.ShapeDtypeStruct((N, D), f32),
               jax.ShapeDtypeStruct((N, 1), f32),
               jax.ShapeDtypeStruct((N, 1), f32)),
)


def _bn_core(aggp, xl, deg, dinv, root, gam, bet):
    out = ((aggp[0] + aggp[1]) * dinv[...]
           + jnp.maximum(xl[...] + root[...], 0.0) / deg[...])
    onesr = jnp.ones((1, N), dtype=f32)
    mean = jnp.dot(onesr, out, preferred_element_type=f32) / N
    sq = jnp.dot(onesr, out * out, preferred_element_type=f32) / N
    var = sq - mean * mean
    return gam[...] * (out - mean) * lax.rsqrt(var + 1e-5) + bet[...]


def _mid_layer_body(aggp, xl, deg, dinv, root, gam, bet, wn, bn, xln):
    hh = jnp.maximum(_bn_core(aggp, xl, deg, dinv, root, gam, bet), 0.0)
    xln[...] = jnp.dot(hh, wn[...], preferred_element_type=f32) + bn[...]


_mid_layer_call = pl.pallas_call(
    _mid_layer_body,
    out_shape=jax.ShapeDtypeStruct((N, D), f32),
)


def _last_layer_body(aggp, xl, deg, dinv, root, gam, bet, h_out):
    h_out[...] = _bn_core(aggp, xl, deg, dinv, root, gam, bet)


_last_layer_call = pl.pallas_call(
    _last_layer_body,
    out_shape=jax.ShapeDtypeStruct((N, D), f32),
)


def kernel(x, edge_index, edge_attr, atom_emb, bond_emb, W, b,
           root_emb, gamma, beta):
    x = x.astype(i32)
    ei = edge_index.astype(i32)
    ea = edge_attr.astype(i32)
    rowe = ei[0]
    cole = ei[1]
    xoff = x + (jnp.arange(AF, dtype=i32) * AV)[None, :]
    xoffT = jnp.pad(xoff.T, ((0, 0), (0, N_PAD - N)))
    eidx = ea[:, 0] + 8 * ea[:, 1] + 64 * ea[:, 2]
    atab = atom_emb.reshape(AF * AV, D)

    btab = _btab_call(bond_emb)
    h0p, degp = _atom_deg_call(atab, xoffT, rowe)
    h0 = h0p[:N]
    xl, deg, dinv2 = _prep_call(h0, W[0], b[0][None, :], degp[:, :N])

    ipk = jnp.stack([rowe.reshape(NW, NCH, ECH),
                     cole.reshape(NW, NCH, ECH),
                     eidx.reshape(NW, NCH, ECH)], axis=2)

    dinvR = jnp.broadcast_to(dinv2, (N, LANES))

    h = None
    for l in range(NLAYERS):
        aggp = _edge_call(xl, ipk, btab[l], dinvR)[:, :N]
        if l < NLAYERS - 1:
            xl = _mid_layer_call(aggp, xl, deg, dinv2, root_emb[l][None, :],
                                 gamma[l][None, :], beta[l][None, :],
                                 W[l + 1], b[l + 1][None, :])
        else:
            h = _last_layer_call(aggp, xl, deg, dinv2, root_emb[l][None, :],
                                 gamma[l][None, :], beta[l][None, :])
    return h
